# baseline (speedup 1.0000x reference)
import jax
import jax.numpy as jnp
from jax.experimental import pallas as pl

_GRID = 8


def _route(lt):
    def row(k):
        return lt[k:k + 1, :]

    part = row(1) > row(0)
    bv = row(2)
    bi = jnp.zeros_like(bv, dtype=jnp.int32)
    t = row(3) > bv
    bi = jnp.where(t, 1, bi)
    bv = jnp.where(t, row(3), bv)
    t = row(4) > bv
    bi = jnp.where(t, 2, bi)
    rv = row(5)
    ri = jnp.zeros_like(bv, dtype=jnp.int32)
    for k in range(1, 8):
        t = row(5 + k) > rv
        ri = jnp.where(t, k, ri)
        rv = jnp.where(t, row(5 + k), rv)
    a0 = row(13) >= row(14)

    branch = jnp.where(bi == 0, 3, jnp.where(bi == 1, ri + 1, jnp.where(a0, 4, 6)))
    return jnp.where(part, branch, 0).astype(jnp.int32)


def _body(x_ref, w_ref, b_ref, o_ref):
    bs = x_ref.shape[0] // 2
    a = x_ref[...].reshape(bs, 2, 128)
    xc = jnp.concatenate([a[:, 0, :], a[:, 1, :]], axis=1)
    l = jnp.dot(xc, w_ref[...], preferred_element_type=jnp.float32)
    l = l + b_ref[...]
    o_ref[0, 0, :] = _route(l.T)[0, :]


def kernel(x, W1, b1, W2, b2, W3r, b3r, W3a, b3a):
    batch = x.shape[0]
    d = x.size // batch
    xr = x.reshape(batch * d // 128, 128)
    W = jnp.concatenate([W1, W2, W3r, W3a], axis=1)
    b = jnp.concatenate([b1, b2, b3r, b3a], axis=0)
    W = jnp.pad(W, ((0, 0), (0, 128 - W.shape[1])))
    b = jnp.pad(b, ((0, 128 - b.shape[0]),)).reshape(1, 128)

    bs = batch // _GRID
    out = pl.pallas_call(
        _body,
        grid=(_GRID,),
        in_specs=[
            pl.BlockSpec((2 * bs, 128), lambda i: (i, 0)),
            pl.BlockSpec((256, 128), lambda i: (0, 0)),
            pl.BlockSpec((1, 128), lambda i: (0, 0)),
        ],
        out_specs=pl.BlockSpec((1, 1, bs), lambda i: (i, 0, 0)),
        out_shape=jax.ShapeDtypeStruct((_GRID, 1, bs), jnp.int32),
    )(xr, W, b)
    return out.reshape(batch)


# speedup vs baseline: 10.8204x; 10.8204x over previous
import jax
import jax.numpy as jnp
from jax.experimental import pallas as pl

_GRID = 8


def _route(lt):
    def row(k):
        return lt[k:k + 1, :]

    part = row(1) > row(0)
    bv = row(2)
    bi = jnp.zeros_like(bv, dtype=jnp.int32)
    t = row(3) > bv
    bi = jnp.where(t, 1, bi)
    bv = jnp.where(t, row(3), bv)
    t = row(4) > bv
    bi = jnp.where(t, 2, bi)
    rv = row(5)
    ri = jnp.zeros_like(bv, dtype=jnp.int32)
    for k in range(1, 8):
        t = row(5 + k) > rv
        ri = jnp.where(t, k, ri)
        rv = jnp.where(t, row(5 + k), rv)
    a0 = row(13) >= row(14)

    branch = jnp.where(bi == 0, 3, jnp.where(bi == 1, ri + 1, jnp.where(a0, 4, 6)))
    return jnp.where(part, branch, 0).astype(jnp.int32)


def _body(xt_ref, wt_ref, bt_ref, o_ref):
    lt = jnp.dot(wt_ref[...], xt_ref[...], preferred_element_type=jnp.float32)
    lt = lt + bt_ref[...]
    o_ref[0, 0, :] = _route(lt)[0, :]


def kernel(x, W1, b1, W2, b2, W3r, b3r, W3a, b3a):
    batch = x.shape[0]
    d = x.size // batch
    xt = jnp.transpose(x, (1, 2, 3, 0)).reshape(d, batch)
    W = jnp.concatenate([W1, W2, W3r, W3a], axis=1)
    b = jnp.concatenate([b1, b2, b3r, b3a], axis=0)
    Wt = jnp.pad(W, ((0, 0), (0, 128 - W.shape[1]))).T
    bt = jnp.pad(b, ((0, 128 - b.shape[0]),)).reshape(128, 1)

    bs = batch // _GRID
    out = pl.pallas_call(
        _body,
        grid=(_GRID,),
        in_specs=[
            pl.BlockSpec((d, bs), lambda i: (0, i)),
            pl.BlockSpec((128, d), lambda i: (0, 0)),
            pl.BlockSpec((128, 1), lambda i: (0, 0)),
        ],
        out_specs=pl.BlockSpec((1, 1, bs), lambda i: (i, 0, 0)),
        out_shape=jax.ShapeDtypeStruct((_GRID, 1, bs), jnp.int32),
    )(xt, Wt, bt)
    return out.reshape(batch)
